# baseline (speedup 1.0000x reference)
import jax
import jax.numpy as jnp
from jax import lax
from jax.experimental import pallas as pl
from jax.experimental.pallas import tpu as pltpu
from jax.experimental.pallas import tpu_sc as plsc

TOTAL = 3276800
VOCAB = 1000000
NC = 2
NS = 16
NW = NC * NS
PER_W = TOTAL // NW
SPC = 9600
NSP = 8
HBC = 12800
NHB = 2
HTOT = NHB * HBC
VPAD = 1000064
SEG = VPAD // NS
SEG_PIECES = (SPC, SPC, SPC, SPC, SPC, SPC, SEG - 6 * SPC)


def _vocab_gather(hashes, table, out, table_sh,
                  sidx0, sidx1, srows0, srows1, hidx, hrows,
                  sisem0, sisem1, sgsem0, sgsem1, sosem0, sosem1,
                  hisem, hgsem, hosem):
    cid = lax.axis_index("c")
    sid = lax.axis_index("s")
    wid = sid * NC + cid
    base = wid * PER_W
    sp_base = base + HTOT
    idx_v = (sidx0, sidx1)
    rows_v = (srows0, srows1)
    isem = (sisem0, sisem1)
    gsem = (sgsem0, sgsem1)
    osem = (sosem0, sosem1)

    icp = [None] * NSP
    for i in range(2):
        icp[i] = pltpu.make_async_copy(
            hashes.at[pl.ds(sp_base + i * SPC, SPC)], idx_v[i], isem[i])
        icp[i].start()
    hicp = pltpu.make_async_copy(hashes.at[pl.ds(base, HBC)], hidx, hisem)
    hicp.start()
    hicp.wait()
    hgcp = pltpu.make_async_copy(table.at[hidx], hrows, hgsem)
    hgcp.start()

    ld = [None, None]
    st = [None, None]
    soff = 0
    for k, sz in enumerate(SEG_PIECES):
        b = k % 2
        if st[b] is not None:
            st[b].wait()
        ld[b] = pltpu.make_async_copy(
            table.at[pl.ds(sid * SEG + soff, sz)],
            rows_v[b].at[pl.ds(0, sz)], gsem[b])
        ld[b].start()
        ld[b].wait()
        st[b] = pltpu.make_async_copy(
            rows_v[b].at[pl.ds(0, sz)],
            table_sh.at[pl.ds(sid * SEG + soff, sz)], osem[b])
        st[b].start()
        soff += sz
    for b in range(2):
        st[b].wait()
    plsc.subcore_barrier()

    ocp = [None] * NSP
    hocp = [None] * NHB
    for i in range(NSP):
        b = i % 2
        off = sp_base + i * SPC
        icp[i].wait()
        if i >= 2:
            ocp[i - 2].wait()
        gcp = pltpu.make_async_copy(table_sh.at[idx_v[b]], rows_v[b], gsem[b])
        gcp.start()
        gcp.wait()
        ocp[i] = pltpu.make_async_copy(
            rows_v[b], out.at[pl.ds(off, SPC)], osem[b])
        ocp[i].start()
        if i + 2 < NSP:
            icp[i + 2] = pltpu.make_async_copy(
                hashes.at[pl.ds(sp_base + (i + 2) * SPC, SPC)],
                idx_v[b], isem[b])
            icp[i + 2].start()
        if i == 3:
            hgcp.wait()
            hocp[0] = pltpu.make_async_copy(
                hrows, out.at[pl.ds(base, HBC)], hosem)
            hocp[0].start()
            hicp = pltpu.make_async_copy(
                hashes.at[pl.ds(base + HBC, HBC)], hidx, hisem)
            hicp.start()
        if i == 4:
            hocp[0].wait()
            hicp.wait()
            hgcp = pltpu.make_async_copy(table.at[hidx], hrows, hgsem)
            hgcp.start()
    hgcp.wait()
    hocp[1] = pltpu.make_async_copy(
        hrows, out.at[pl.ds(base + HBC, HBC)], hosem)
    hocp[1].start()
    ocp[NSP - 2].wait()
    ocp[NSP - 1].wait()
    hocp[1].wait()


def kernel(token_hashes, start_ids, end_ids, vocab_table):
    table_p = jnp.pad(vocab_table, (0, VPAD - VOCAB))
    mesh = plsc.VectorSubcoreMesh(core_axis_name="c", subcore_axis_name="s")
    gather = pl.kernel(
        _vocab_gather,
        out_type=jax.ShapeDtypeStruct((TOTAL,), jnp.float32),
        mesh=mesh,
        scratch_types=[
            pltpu.VMEM_SHARED((VPAD,), jnp.float32),
            pltpu.VMEM((SPC,), jnp.int32),
            pltpu.VMEM((SPC,), jnp.int32),
            pltpu.VMEM((SPC,), jnp.float32),
            pltpu.VMEM((SPC,), jnp.float32),
            pltpu.VMEM((HBC,), jnp.int32),
            pltpu.VMEM((HBC,), jnp.float32),
            pltpu.SemaphoreType.DMA,
            pltpu.SemaphoreType.DMA,
            pltpu.SemaphoreType.DMA,
            pltpu.SemaphoreType.DMA,
            pltpu.SemaphoreType.DMA,
            pltpu.SemaphoreType.DMA,
            pltpu.SemaphoreType.DMA,
            pltpu.SemaphoreType.DMA,
            pltpu.SemaphoreType.DMA,
        ],
    )
    token_ids = gather(token_hashes, table_p)
    return (token_ids, start_ids, end_ids)


# speedup vs baseline: 1.1659x; 1.1659x over previous
import jax
import jax.numpy as jnp
from jax import lax
from jax.experimental import pallas as pl
from jax.experimental.pallas import tpu as pltpu
from jax.experimental.pallas import tpu_sc as plsc

TOTAL = 3276800
VOCAB = 1000000
NC = 2
NS = 16
NW = NC * NS
PER_W = TOTAL // NW
CHUNK = 12800
NCHUNK = PER_W // CHUNK
VPAD = 1000064
SEG = VPAD // NS
SEG_PIECES = (CHUNK, CHUNK, CHUNK, CHUNK, SEG - 4 * CHUNK)


def _vocab_gather(hashes, table, out, table_sh,
                  idx0, idx1, rows0, rows1,
                  isem0, isem1, gsem0, gsem1, osem0, osem1):
    cid = lax.axis_index("c")
    sid = lax.axis_index("s")
    wid = sid * NC + cid
    base = wid * PER_W
    idx_v = (idx0, idx1)
    rows_v = (rows0, rows1)
    isem = (isem0, isem1)
    gsem = (gsem0, gsem1)
    osem = (osem0, osem1)

    icp = [None] * NCHUNK
    for i in range(2):
        icp[i] = pltpu.make_async_copy(
            hashes.at[pl.ds(base + i * CHUNK, CHUNK)], idx_v[i], isem[i])
        icp[i].start()

    ld = [None, None]
    st = [None, None]
    soff = 0
    for k, sz in enumerate(SEG_PIECES):
        b = k % 2
        if st[b] is not None:
            st[b].wait()
        ld[b] = pltpu.make_async_copy(
            table.at[pl.ds(sid * SEG + soff, sz)],
            rows_v[b].at[pl.ds(0, sz)], gsem[b])
        ld[b].start()
        ld[b].wait()
        st[b] = pltpu.make_async_copy(
            rows_v[b].at[pl.ds(0, sz)],
            table_sh.at[pl.ds(sid * SEG + soff, sz)], osem[b])
        st[b].start()
        soff += sz
    for b in range(2):
        if st[b] is not None:
            st[b].wait()
    plsc.subcore_barrier()

    ocp = [None] * NCHUNK
    for i in range(NCHUNK):
        b = i % 2
        off = base + i * CHUNK
        icp[i].wait()
        if i >= 2:
            ocp[i - 2].wait()
        gcp = pltpu.make_async_copy(table_sh.at[idx_v[b]], rows_v[b], gsem[b])
        gcp.start()
        gcp.wait()
        ocp[i] = pltpu.make_async_copy(
            rows_v[b], out.at[pl.ds(off, CHUNK)], osem[b])
        ocp[i].start()
        if i + 2 < NCHUNK:
            icp[i + 2] = pltpu.make_async_copy(
                hashes.at[pl.ds(base + (i + 2) * CHUNK, CHUNK)],
                idx_v[b], isem[b])
            icp[i + 2].start()
    ocp[NCHUNK - 2].wait()
    ocp[NCHUNK - 1].wait()


def kernel(token_hashes, start_ids, end_ids, vocab_table):
    table_p = jnp.pad(vocab_table, (0, VPAD - VOCAB))
    mesh = plsc.VectorSubcoreMesh(core_axis_name="c", subcore_axis_name="s")
    gather = pl.kernel(
        _vocab_gather,
        out_type=jax.ShapeDtypeStruct((TOTAL,), jnp.float32),
        mesh=mesh,
        scratch_types=[
            pltpu.VMEM_SHARED((VPAD,), jnp.float32),
            pltpu.VMEM((CHUNK,), jnp.int32),
            pltpu.VMEM((CHUNK,), jnp.int32),
            pltpu.VMEM((CHUNK,), jnp.float32),
            pltpu.VMEM((CHUNK,), jnp.float32),
            pltpu.SemaphoreType.DMA,
            pltpu.SemaphoreType.DMA,
            pltpu.SemaphoreType.DMA,
            pltpu.SemaphoreType.DMA,
            pltpu.SemaphoreType.DMA,
            pltpu.SemaphoreType.DMA,
        ],
    )
    token_ids = gather(token_hashes, table_p)
    return (token_ids, start_ids, end_ids)


# speedup vs baseline: 1.2390x; 1.0627x over previous
import jax
import jax.numpy as jnp
from jax import lax
from jax.experimental import pallas as pl
from jax.experimental.pallas import tpu as pltpu
from jax.experimental.pallas import tpu_sc as plsc

TOTAL = 3276800
VOCAB = 1000000
NC = 2
NS = 16
NW = NC * NS
PER_W = TOTAL // NW
CHUNK = 10240
NCHUNK = PER_W // CHUNK
SEG = 62504
SEG_PIECES = (CHUNK, CHUNK, CHUNK, CHUNK, CHUNK, CHUNK, SEG - 6 * CHUNK)


def _vocab_gather(hashes, starts, ends, table,
                  out, out_s, out_e, table_sh,
                  idx0, idx1, rows0, rows1, pb0, pb1,
                  isem0, isem1, gsem0, gsem1, osem0, osem1,
                  plsem0, plsem1, pssem0, pssem1):
    cid = lax.axis_index("c")
    sid = lax.axis_index("s")
    wid = sid * NC + cid
    base = wid * PER_W
    idx_v = (idx0, idx1)
    rows_v = (rows0, rows1)
    isem = (isem0, isem1)
    gsem = (gsem0, gsem1)
    osem = (osem0, osem1)
    pb = (pb0, pb1)
    plsem = (plsem0, plsem1)
    pssem = (pssem0, pssem1)
    psrc = (starts, ends)
    pdst = (out_s, out_e)

    icp = [None] * NCHUNK
    for i in range(2):
        icp[i] = pltpu.make_async_copy(
            hashes.at[pl.ds(base + i * CHUNK, CHUNK)], idx_v[i], isem[i])
        icp[i].start()

    seg_off = lax.min(sid * SEG, VOCAB - SEG)
    ld = [None, None]
    st = [None, None]
    soff = 0
    for k, sz in enumerate(SEG_PIECES):
        b = k % 2
        if st[b] is not None:
            st[b].wait()
        ld[b] = pltpu.make_async_copy(
            table.at[pl.ds(seg_off + soff, sz)],
            rows_v[b].at[pl.ds(0, sz)], gsem[b])
        ld[b].start()
        ld[b].wait()
        st[b] = pltpu.make_async_copy(
            rows_v[b].at[pl.ds(0, sz)],
            table_sh.at[pl.ds(seg_off + soff, sz)], osem[b])
        st[b].start()
        soff += sz
    for b in range(2):
        st[b].wait()
    plsc.subcore_barrier()

    ocp = [None] * NCHUNK
    pld = [None, None]
    pst = [None, None]
    for i in range(NCHUNK):
        b = i % 2
        off = base + i * CHUNK
        for a in range(2):
            if pst[a] is not None:
                pst[a].wait()
            pld[a] = pltpu.make_async_copy(
                psrc[a].at[pl.ds(off, CHUNK)], pb[a], plsem[a])
            pld[a].start()
        icp[i].wait()
        if i >= 2:
            ocp[i - 2].wait()
        gcp = pltpu.make_async_copy(table_sh.at[idx_v[b]], rows_v[b], gsem[b])
        gcp.start()
        gcp.wait()
        ocp[i] = pltpu.make_async_copy(
            rows_v[b], out.at[pl.ds(off, CHUNK)], osem[b])
        ocp[i].start()
        if i + 2 < NCHUNK:
            icp[i + 2] = pltpu.make_async_copy(
                hashes.at[pl.ds(base + (i + 2) * CHUNK, CHUNK)],
                idx_v[b], isem[b])
            icp[i + 2].start()
        for a in range(2):
            pld[a].wait()
            pst[a] = pltpu.make_async_copy(
                pb[a], pdst[a].at[pl.ds(off, CHUNK)], pssem[a])
            pst[a].start()
    ocp[NCHUNK - 2].wait()
    ocp[NCHUNK - 1].wait()
    pst[0].wait()
    pst[1].wait()


def kernel(token_hashes, start_ids, end_ids, vocab_table):
    mesh = plsc.VectorSubcoreMesh(core_axis_name="c", subcore_axis_name="s")
    gather = pl.kernel(
        _vocab_gather,
        out_type=(
            jax.ShapeDtypeStruct((TOTAL,), jnp.float32),
            jax.ShapeDtypeStruct((TOTAL,), jnp.int32),
            jax.ShapeDtypeStruct((TOTAL,), jnp.int32),
        ),
        mesh=mesh,
        scratch_types=[
            pltpu.VMEM_SHARED((VOCAB,), jnp.float32),
            pltpu.VMEM((CHUNK,), jnp.int32),
            pltpu.VMEM((CHUNK,), jnp.int32),
            pltpu.VMEM((CHUNK,), jnp.float32),
            pltpu.VMEM((CHUNK,), jnp.float32),
            pltpu.VMEM((CHUNK,), jnp.int32),
            pltpu.VMEM((CHUNK,), jnp.int32),
            pltpu.SemaphoreType.DMA,
            pltpu.SemaphoreType.DMA,
            pltpu.SemaphoreType.DMA,
            pltpu.SemaphoreType.DMA,
            pltpu.SemaphoreType.DMA,
            pltpu.SemaphoreType.DMA,
            pltpu.SemaphoreType.DMA,
            pltpu.SemaphoreType.DMA,
            pltpu.SemaphoreType.DMA,
            pltpu.SemaphoreType.DMA,
        ],
    )
    token_ids, sids, eids = gather(token_hashes, start_ids, end_ids,
                                   vocab_table)
    return (token_ids, sids, eids)


# speedup vs baseline: 1.2653x; 1.0212x over previous
import jax
import jax.numpy as jnp
from jax import lax
from jax.experimental import pallas as pl
from jax.experimental.pallas import tpu as pltpu
from jax.experimental.pallas import tpu_sc as plsc

TOTAL = 3276800
VOCAB = 1000000
NC = 2
NS = 16
NW = NC * NS
PER_W = TOTAL // NW
CHUNK = 10240
NCHUNK = PER_W // CHUNK
SEG = 62504
SEG_PIECES = (CHUNK, CHUNK, CHUNK, CHUNK, CHUNK, CHUNK, SEG - 6 * CHUNK)


def _vocab_gather(hashes, starts, table,
                  out, out_s, table_sh,
                  idx0, idx1, rows0, rows1, pb0, pb1,
                  isem0, isem1, gsem0, gsem1, osem0, osem1,
                  plsem0, plsem1, pssem0, pssem1):
    cid = lax.axis_index("c")
    sid = lax.axis_index("s")
    wid = sid * NC + cid
    base = wid * PER_W
    idx_v = (idx0, idx1)
    rows_v = (rows0, rows1)
    isem = (isem0, isem1)
    gsem = (gsem0, gsem1)
    osem = (osem0, osem1)
    pb = (pb0, pb1)
    plsem = (plsem0, plsem1)
    pssem = (pssem0, pssem1)

    icp = [None] * NCHUNK
    for i in range(2):
        icp[i] = pltpu.make_async_copy(
            hashes.at[pl.ds(base + i * CHUNK, CHUNK)], idx_v[i], isem[i])
        icp[i].start()

    seg_off = lax.min(sid * SEG, VOCAB - SEG)
    ld = [None, None]
    st = [None, None]
    soff = 0
    for k, sz in enumerate(SEG_PIECES):
        b = k % 2
        if st[b] is not None:
            st[b].wait()
        ld[b] = pltpu.make_async_copy(
            table.at[pl.ds(seg_off + soff, sz)],
            rows_v[b].at[pl.ds(0, sz)], gsem[b])
        ld[b].start()
        ld[b].wait()
        st[b] = pltpu.make_async_copy(
            rows_v[b].at[pl.ds(0, sz)],
            table_sh.at[pl.ds(seg_off + soff, sz)], osem[b])
        st[b].start()
        soff += sz
    for b in range(2):
        st[b].wait()
    plsc.subcore_barrier()

    ocp = [None] * NCHUNK
    pld = [None, None]
    pst = [None, None]
    for i in range(NCHUNK):
        b = i % 2
        off = base + i * CHUNK
        if pst[b] is not None:
            pst[b].wait()
        pld[b] = pltpu.make_async_copy(
            starts.at[pl.ds(off, CHUNK)], pb[b], plsem[b])
        pld[b].start()
        icp[i].wait()
        if i >= 2:
            ocp[i - 2].wait()
        gcp = pltpu.make_async_copy(table_sh.at[idx_v[b]], rows_v[b], gsem[b])
        gcp.start()
        gcp.wait()
        ocp[i] = pltpu.make_async_copy(
            rows_v[b], out.at[pl.ds(off, CHUNK)], osem[b])
        ocp[i].start()
        if i + 2 < NCHUNK:
            icp[i + 2] = pltpu.make_async_copy(
                hashes.at[pl.ds(base + (i + 2) * CHUNK, CHUNK)],
                idx_v[b], isem[b])
            icp[i + 2].start()
        pld[b].wait()
        pst[b] = pltpu.make_async_copy(
            pb[b], out_s.at[pl.ds(off, CHUNK)], pssem[b])
        pst[b].start()
    ocp[NCHUNK - 2].wait()
    ocp[NCHUNK - 1].wait()
    pst[0].wait()
    pst[1].wait()


def kernel(token_hashes, start_ids, end_ids, vocab_table):
    mesh = plsc.VectorSubcoreMesh(core_axis_name="c", subcore_axis_name="s")
    gather = pl.kernel(
        _vocab_gather,
        out_type=(
            jax.ShapeDtypeStruct((TOTAL,), jnp.float32),
            jax.ShapeDtypeStruct((TOTAL,), jnp.int32),
        ),
        mesh=mesh,
        scratch_types=[
            pltpu.VMEM_SHARED((VOCAB,), jnp.float32),
            pltpu.VMEM((CHUNK,), jnp.int32),
            pltpu.VMEM((CHUNK,), jnp.int32),
            pltpu.VMEM((CHUNK,), jnp.float32),
            pltpu.VMEM((CHUNK,), jnp.float32),
            pltpu.VMEM((CHUNK,), jnp.int32),
            pltpu.VMEM((CHUNK,), jnp.int32),
            pltpu.SemaphoreType.DMA,
            pltpu.SemaphoreType.DMA,
            pltpu.SemaphoreType.DMA,
            pltpu.SemaphoreType.DMA,
            pltpu.SemaphoreType.DMA,
            pltpu.SemaphoreType.DMA,
            pltpu.SemaphoreType.DMA,
            pltpu.SemaphoreType.DMA,
            pltpu.SemaphoreType.DMA,
            pltpu.SemaphoreType.DMA,
        ],
    )
    token_ids, sids = gather(token_hashes, start_ids, vocab_table)
    return (token_ids, sids, end_ids)
